# baseline (speedup 1.0000x reference)
import functools

import jax
import jax.numpy as jnp
from jax import lax
from jax.experimental import pallas as pl
from jax.experimental.pallas import tpu as pltpu
from jax.experimental.pallas import tpu_sc as plsc

_INFO = plsc.get_sparse_core_info()
_NS = _INFO.num_subcores
_NCHUNK = 4


@jax.jit
def _gather_sc(x, idx):
    B = idx.shape[0]
    D = x.shape[1]
    b_per_w = B // _NS
    c = b_per_w // _NCHUNK

    mesh = plsc.VectorSubcoreMesh(
        core_axis_name="c", subcore_axis_name="s", num_cores=1
    )

    @functools.partial(
        pl.kernel,
        mesh=mesh,
        out_type=jax.ShapeDtypeStruct((B, D), jnp.float32),
        scratch_types=[
            pltpu.VMEM((b_per_w,), jnp.int32),
            pltpu.VMEM((b_per_w, D), jnp.float32),
            [pltpu.SemaphoreType.DMA] * _NCHUNK,
            [pltpu.SemaphoreType.DMA] * _NCHUNK,
        ],
    )
    def k(x_hbm, idx_hbm, out_hbm, idx_v, rows_v, gsems, ssems):
        base = lax.axis_index("s") * b_per_w
        pltpu.sync_copy(idx_hbm.at[pl.ds(base, b_per_w)], idx_v)
        gathers = []
        for j in range(_NCHUNK):
            gathers.append(
                pltpu.async_copy(
                    x_hbm.at[idx_v.at[pl.ds(j * c, c)]],
                    rows_v.at[pl.ds(j * c, c)],
                    gsems[j],
                )
            )
        stores = []
        for j in range(_NCHUNK):
            gathers[j].wait()
            stores.append(
                pltpu.async_copy(
                    rows_v.at[pl.ds(j * c, c)],
                    out_hbm.at[pl.ds(base + j * c, c)],
                    ssems[j],
                )
            )
        for s in stores:
            s.wait()

    return k(x, idx)


def kernel(x, node_offsets):
    return _gather_sc(x, node_offsets.astype(jnp.int32))


# speedup vs baseline: 1.0127x; 1.0127x over previous
import functools

import jax
import jax.numpy as jnp
from jax import lax
from jax.experimental import pallas as pl
from jax.experimental.pallas import tpu as pltpu
from jax.experimental.pallas import tpu_sc as plsc

_INFO = plsc.get_sparse_core_info()
_NS = _INFO.num_subcores


@jax.jit
def _gather_sc(x, idx):
    B = idx.shape[0]
    D = x.shape[1]
    b_per_w = B // _NS

    mesh = plsc.VectorSubcoreMesh(
        core_axis_name="c", subcore_axis_name="s", num_cores=1
    )

    @functools.partial(
        pl.kernel,
        mesh=mesh,
        out_type=jax.ShapeDtypeStruct((B, D), jnp.float32),
        scratch_types=[
            pltpu.VMEM((b_per_w,), jnp.int32),
            pltpu.VMEM((b_per_w, D), jnp.float32),
            pltpu.SemaphoreType.DMA,
            pltpu.SemaphoreType.DMA,
            pltpu.SemaphoreType.DMA,
        ],
    )
    def k(x_hbm, idx_hbm, out_hbm, idx_v, rows_v, g0s, g1s, s0s):
        half = b_per_w // 2
        base = lax.axis_index("s") * b_per_w
        pltpu.sync_copy(idx_hbm.at[pl.ds(base, b_per_w)], idx_v)
        g0 = pltpu.async_copy(
            x_hbm.at[idx_v.at[pl.ds(0, half)]], rows_v.at[pl.ds(0, half)], g0s
        )
        g1 = pltpu.async_copy(
            x_hbm.at[idx_v.at[pl.ds(half, half)]], rows_v.at[pl.ds(half, half)], g1s
        )
        g0.wait()
        s0 = pltpu.async_copy(
            rows_v.at[pl.ds(0, half)], out_hbm.at[pl.ds(base, half)], s0s
        )
        g1.wait()
        pltpu.sync_copy(
            rows_v.at[pl.ds(half, half)], out_hbm.at[pl.ds(base + half, half)]
        )
        s0.wait()

    return k(x, idx)


def kernel(x, node_offsets):
    return _gather_sc(x, node_offsets.astype(jnp.int32))
